# Initial kernel scaffold; baseline (speedup 1.0000x reference)
import jax
import jax.numpy as jnp
from jax import lax
from jax.experimental import pallas as pl
from jax.experimental.pallas import tpu as pltpu
from jax.experimental.pallas import tpu_sc as plsc

N_NODES = 10000
DIM = 128
N_EDGES = 320000

NC = 2
NS = 16
NW = NC * NS
CHUNK = 128
CHUNKS_PER_W = -(-N_EDGES // (NW * CHUNK))
EDGES_PER_W = CHUNKS_PER_W * CHUNK
E_PAD = NW * EDGES_PER_W
NP = 10240
ROWS_PER_W = NP // NS

_MESH = plsc.VectorSubcoreMesh(
    core_axis_name="c", subcore_axis_name="s", num_cores=NC, num_subcores=NS
)


def _sc_edge_pass(with_counts, xl, src, dst, zeros_d, zeros_c, ones_c):
    out_types = [jax.ShapeDtypeStruct((NC, NP, DIM), jnp.float32)]
    scratch = [
        pltpu.VMEM((CHUNK,), jnp.int32),
        pltpu.VMEM((CHUNK,), jnp.int32),
        pltpu.VMEM((CHUNK, DIM), jnp.float32),
        pltpu.VMEM_SHARED((NP, DIM), jnp.float32),
        pltpu.SemaphoreType.DMA,
    ]
    if with_counts:
        out_types.append(jax.ShapeDtypeStruct((NC, NP, 16), jnp.float32))
        scratch += [
            pltpu.VMEM((CHUNK, 16), jnp.float32),
            pltpu.VMEM_SHARED((NP, 16), jnp.float32),
        ]

    def body(xl_hbm, src_hbm, dst_hbm, zd_hbm, zc_hbm, on_hbm,
             out_hbm, *rest):
        if with_counts:
            cnt_hbm, srcv, dstv, rows, acc, sem, onesv, cacc = rest
        else:
            srcv, dstv, rows, acc, sem = rest
        c = lax.axis_index("c")
        s = lax.axis_index("s")
        r0 = s * ROWS_PER_W
        pltpu.sync_copy(zd_hbm, acc.at[pl.ds(r0, ROWS_PER_W)])
        if with_counts:
            pltpu.sync_copy(zc_hbm, cacc.at[pl.ds(r0, ROWS_PER_W)])
            pltpu.sync_copy(on_hbm, onesv)
        plsc.subcore_barrier()

        base = (c * NS + s) * EDGES_PER_W

        @pl.loop(0, CHUNKS_PER_W)
        def _(j):
            e0 = base + j * CHUNK
            pltpu.sync_copy(src_hbm.at[pl.ds(e0, CHUNK)], srcv)
            pltpu.sync_copy(dst_hbm.at[pl.ds(e0, CHUNK)], dstv)
            pltpu.async_copy(xl_hbm.at[srcv], rows, sem).wait()
            pltpu.sync_copy(rows, acc.at[dstv], add=True)
            if with_counts:
                pltpu.sync_copy(onesv, cacc.at[dstv], add=True)

        plsc.subcore_barrier()
        pltpu.sync_copy(acc.at[pl.ds(r0, ROWS_PER_W)],
                        out_hbm.at[c, pl.ds(r0, ROWS_PER_W)])
        if with_counts:
            pltpu.sync_copy(cacc.at[pl.ds(r0, ROWS_PER_W)],
                            cnt_hbm.at[c, pl.ds(r0, ROWS_PER_W)])

    k = pl.kernel(body, out_type=tuple(out_types), mesh=_MESH,
                  scratch_types=scratch)
    return k(xl, src, dst, zeros_d, zeros_c, ones_c)


_BR = 1024


def _tc_linear(x, W, b):
    def body(x_ref, w_ref, b_ref, o_ref):
        o_ref[...] = lax.dot_general(
            x_ref[...], w_ref[...], (((1,), (1,)), ((), ())),
            preferred_element_type=jnp.float32) + b_ref[...]

    return pl.pallas_call(
        body,
        grid=(NP // _BR,),
        in_specs=[
            pl.BlockSpec((_BR, DIM), lambda i: (i, 0)),
            pl.BlockSpec((DIM, DIM), lambda i: (0, 0)),
            pl.BlockSpec((1, DIM), lambda i: (0, 0)),
        ],
        out_specs=pl.BlockSpec((_BR, DIM), lambda i: (i, 0)),
        out_shape=jax.ShapeDtypeStruct((NP, DIM), jnp.float32),
    )(x, W, b)


def _mean_from_partials(p_ref, cp_ref):
    cnt = cp_ref[0, :, 0] + cp_ref[1, :, 0]
    inv = 1.0 / jnp.clip(cnt, 1.0)
    return (p_ref[0] + p_ref[1]) * inv[:, None]


def _tc_combine_relu_linear(p, cp, W, b):
    def body(p_ref, cp_ref, w_ref, b_ref, o_ref):
        h = jnp.maximum(_mean_from_partials(p_ref, cp_ref), 0.0)
        o_ref[...] = lax.dot_general(
            h, w_ref[...], (((1,), (1,)), ((), ())),
            preferred_element_type=jnp.float32) + b_ref[...]

    return pl.pallas_call(
        body,
        grid=(NP // _BR,),
        in_specs=[
            pl.BlockSpec((NC, _BR, DIM), lambda i: (0, i, 0)),
            pl.BlockSpec((NC, _BR, 16), lambda i: (0, i, 0)),
            pl.BlockSpec((DIM, DIM), lambda i: (0, 0)),
            pl.BlockSpec((1, DIM), lambda i: (0, 0)),
        ],
        out_specs=pl.BlockSpec((_BR, DIM), lambda i: (i, 0)),
        out_shape=jax.ShapeDtypeStruct((NP, DIM), jnp.float32),
    )(p, cp, W, b)


def _tc_combine_logsoftmax(p, cp):
    def body(p_ref, cp_ref, h_ref, ls_ref):
        h = _mean_from_partials(p_ref, cp_ref)
        h_ref[...] = h
        m = jnp.max(h, axis=1, keepdims=True)
        lse = jnp.log(jnp.sum(jnp.exp(h - m), axis=1, keepdims=True)) + m
        ls_ref[...] = h - lse

    return pl.pallas_call(
        body,
        grid=(NP // _BR,),
        in_specs=[
            pl.BlockSpec((NC, _BR, DIM), lambda i: (0, i, 0)),
            pl.BlockSpec((NC, _BR, 16), lambda i: (0, i, 0)),
        ],
        out_specs=[
            pl.BlockSpec((_BR, DIM), lambda i: (i, 0)),
            pl.BlockSpec((_BR, DIM), lambda i: (i, 0)),
        ],
        out_shape=[
            jax.ShapeDtypeStruct((NP, DIM), jnp.float32),
            jax.ShapeDtypeStruct((NP, DIM), jnp.float32),
        ],
    )(p, cp)


def kernel(x, edge_index, W1, b1, W2, b2):
    x = x.astype(jnp.float32)
    xp = jnp.zeros((NP, DIM), jnp.float32).at[:N_NODES].set(x)
    src = edge_index[0].astype(jnp.int32)
    dst = edge_index[1].astype(jnp.int32)
    pad = E_PAD - N_EDGES
    srcp = jnp.concatenate([src, jnp.zeros((pad,), jnp.int32)])
    dstp = jnp.concatenate([dst, jnp.full((pad,), N_NODES, jnp.int32)])
    zeros_d = jnp.zeros((ROWS_PER_W, DIM), jnp.float32)
    zeros_c = jnp.zeros((ROWS_PER_W, 16), jnp.float32)
    ones_c = jnp.ones((CHUNK, 16), jnp.float32)
    b1r = b1.reshape(1, DIM)
    b2r = b2.reshape(1, DIM)

    xl1 = _tc_linear(xp, W1, b1r)
    p1, cp = _sc_edge_pass(True, xl1, srcp, dstp, zeros_d, zeros_c, ones_c)
    xl2 = _tc_combine_relu_linear(p1, cp, W2, b2r)
    (p2,) = _sc_edge_pass(False, xl2, srcp, dstp, zeros_d, zeros_c, ones_c)
    h2, ls = _tc_combine_logsoftmax(p2, cp)
    return h2[:N_NODES], ls[:N_NODES]


# speedup vs baseline: 5.4330x; 5.4330x over previous
import jax
import jax.numpy as jnp
from jax import lax
from jax.experimental import pallas as pl
from jax.experimental.pallas import tpu as pltpu
from jax.experimental.pallas import tpu_sc as plsc

N_NODES = 10000
DIM = 128
N_EDGES = 320000

NC = 2
NS = 16
NW = NC * NS
CHUNK = 128
CHUNKS_PER_W = -(-N_EDGES // (NW * CHUNK))
EDGES_PER_W = CHUNKS_PER_W * CHUNK
E_PAD = NW * EDGES_PER_W
NP = 10240
ROWS_PER_W = NP // NS

_MESH = plsc.VectorSubcoreMesh(
    core_axis_name="c", subcore_axis_name="s", num_cores=NC, num_subcores=NS
)


def _sc_edge_pass(xl, src, dst, zeros_d):
    def body(xl_hbm, src_hbm, dst_hbm, zd_hbm, out_hbm,
             srcv, dstv, rows, acc, sem):
        c = lax.axis_index("c")
        s = lax.axis_index("s")
        r0 = s * ROWS_PER_W
        pltpu.sync_copy(zd_hbm, acc.at[pl.ds(r0, ROWS_PER_W)])
        plsc.subcore_barrier()

        base = (c * NS + s) * EDGES_PER_W

        @pl.loop(0, CHUNKS_PER_W)
        def _(j):
            e0 = base + j * CHUNK
            pltpu.sync_copy(src_hbm.at[pl.ds(e0, CHUNK)], srcv)
            pltpu.sync_copy(dst_hbm.at[pl.ds(e0, CHUNK)], dstv)
            pltpu.async_copy(xl_hbm.at[srcv], rows, sem).wait()
            pltpu.sync_copy(rows, acc.at[dstv], add=True)

        plsc.subcore_barrier()
        pltpu.sync_copy(acc.at[pl.ds(r0, ROWS_PER_W)],
                        out_hbm.at[c, pl.ds(r0, ROWS_PER_W)])

    k = pl.kernel(
        body,
        out_type=jax.ShapeDtypeStruct((NC, NP, DIM), jnp.float32),
        mesh=_MESH,
        scratch_types=[
            pltpu.VMEM((CHUNK,), jnp.int32),
            pltpu.VMEM((CHUNK,), jnp.int32),
            pltpu.VMEM((CHUNK, DIM), jnp.float32),
            pltpu.VMEM_SHARED((NP, DIM), jnp.float32),
            pltpu.SemaphoreType.DMA,
        ],
    )
    return k(xl, src, dst, zeros_d)


def _sc_count_pass(dst, zeros_d, ones_d):
    def body(dst_hbm, zd_hbm, on_hbm, out_hbm, dstv, onesv, acc, sem):
        c = lax.axis_index("c")
        s = lax.axis_index("s")
        r0 = s * ROWS_PER_W
        pltpu.sync_copy(zd_hbm, acc.at[pl.ds(r0, ROWS_PER_W)])
        pltpu.sync_copy(on_hbm, onesv)
        plsc.subcore_barrier()

        base = (c * NS + s) * EDGES_PER_W

        @pl.loop(0, CHUNKS_PER_W)
        def _(j):
            pltpu.sync_copy(dst_hbm.at[pl.ds(base + j * CHUNK, CHUNK)], dstv)
            pltpu.sync_copy(onesv, acc.at[dstv], add=True)

        plsc.subcore_barrier()
        pltpu.sync_copy(acc.at[pl.ds(r0, ROWS_PER_W)],
                        out_hbm.at[c, pl.ds(r0, ROWS_PER_W)])

    k = pl.kernel(
        body,
        out_type=jax.ShapeDtypeStruct((NC, NP, DIM), jnp.float32),
        mesh=_MESH,
        scratch_types=[
            pltpu.VMEM((CHUNK,), jnp.int32),
            pltpu.VMEM((CHUNK, DIM), jnp.float32),
            pltpu.VMEM_SHARED((NP, DIM), jnp.float32),
            pltpu.SemaphoreType.DMA,
        ],
    )
    return k(dst, zeros_d, ones_d)


_BR = 1024


def _tc_linear(x, W, b):
    def body(x_ref, w_ref, b_ref, o_ref):
        o_ref[...] = lax.dot_general(
            x_ref[...], w_ref[...], (((1,), (1,)), ((), ())),
            preferred_element_type=jnp.float32) + b_ref[...]

    return pl.pallas_call(
        body,
        grid=(NP // _BR,),
        in_specs=[
            pl.BlockSpec((_BR, DIM), lambda i: (i, 0)),
            pl.BlockSpec((DIM, DIM), lambda i: (0, 0)),
            pl.BlockSpec((1, DIM), lambda i: (0, 0)),
        ],
        out_specs=pl.BlockSpec((_BR, DIM), lambda i: (i, 0)),
        out_shape=jax.ShapeDtypeStruct((NP, DIM), jnp.float32),
    )(x, W, b)


def _mean_from_partials(p_ref, cp_ref):
    cnt = cp_ref[0, :, 0] + cp_ref[1, :, 0]
    inv = 1.0 / jnp.clip(cnt, 1.0)
    return (p_ref[0] + p_ref[1]) * inv[:, None]


def _tc_combine_relu_linear(p, cp, W, b):
    def body(p_ref, cp_ref, w_ref, b_ref, o_ref):
        h = jnp.maximum(_mean_from_partials(p_ref, cp_ref), 0.0)
        o_ref[...] = lax.dot_general(
            h, w_ref[...], (((1,), (1,)), ((), ())),
            preferred_element_type=jnp.float32) + b_ref[...]

    return pl.pallas_call(
        body,
        grid=(NP // _BR,),
        in_specs=[
            pl.BlockSpec((NC, _BR, DIM), lambda i: (0, i, 0)),
            pl.BlockSpec((NC, _BR, DIM), lambda i: (0, i, 0)),
            pl.BlockSpec((DIM, DIM), lambda i: (0, 0)),
            pl.BlockSpec((1, DIM), lambda i: (0, 0)),
        ],
        out_specs=pl.BlockSpec((_BR, DIM), lambda i: (i, 0)),
        out_shape=jax.ShapeDtypeStruct((NP, DIM), jnp.float32),
    )(p, cp, W, b)


def _tc_combine_logsoftmax(p, cp):
    def body(p_ref, cp_ref, h_ref, ls_ref):
        h = _mean_from_partials(p_ref, cp_ref)
        h_ref[...] = h
        m = jnp.max(h, axis=1, keepdims=True)
        lse = jnp.log(jnp.sum(jnp.exp(h - m), axis=1, keepdims=True)) + m
        ls_ref[...] = h - lse

    return pl.pallas_call(
        body,
        grid=(NP // _BR,),
        in_specs=[
            pl.BlockSpec((NC, _BR, DIM), lambda i: (0, i, 0)),
            pl.BlockSpec((NC, _BR, DIM), lambda i: (0, i, 0)),
        ],
        out_specs=[
            pl.BlockSpec((_BR, DIM), lambda i: (i, 0)),
            pl.BlockSpec((_BR, DIM), lambda i: (i, 0)),
        ],
        out_shape=[
            jax.ShapeDtypeStruct((NP, DIM), jnp.float32),
            jax.ShapeDtypeStruct((NP, DIM), jnp.float32),
        ],
    )(p, cp)


def kernel(x, edge_index, W1, b1, W2, b2):
    x = x.astype(jnp.float32)
    xp = jnp.zeros((NP, DIM), jnp.float32).at[:N_NODES].set(x)
    src = edge_index[0].astype(jnp.int32)
    dst = edge_index[1].astype(jnp.int32)
    pad = E_PAD - N_EDGES
    pad_iota = jnp.arange(pad, dtype=jnp.int32)
    srcp = jnp.concatenate([src, pad_iota % N_NODES])
    dstp = jnp.concatenate([dst, N_NODES + pad_iota % (NP - N_NODES)])
    zeros_d = jnp.zeros((ROWS_PER_W, DIM), jnp.float32)
    ones_d = jnp.ones((CHUNK, DIM), jnp.float32)
    b1r = b1.reshape(1, DIM)
    b2r = b2.reshape(1, DIM)

    cp = _sc_count_pass(dstp, zeros_d, ones_d)
    xl1 = _tc_linear(xp, W1, b1r)
    p1 = _sc_edge_pass(xl1, srcp, dstp, zeros_d)
    xl2 = _tc_combine_relu_linear(p1, cp, W2, b2r)
    p2 = _sc_edge_pass(xl2, srcp, dstp, zeros_d)
    h2, ls = _tc_combine_logsoftmax(p2, cp)
    return h2[:N_NODES], ls[:N_NODES]


# speedup vs baseline: 9.7302x; 1.7910x over previous
import jax
import jax.numpy as jnp
from jax import lax
from jax.experimental import pallas as pl
from jax.experimental.pallas import tpu as pltpu
from jax.experimental.pallas import tpu_sc as plsc

N_NODES = 10000
DIM = 128
N_EDGES = 320000

NC = 2
NS = 16
NW = NC * NS
CHUNK = 128
NBUF = 2
CHUNKS_PER_W = 80
NHALF = 2
CH_H = CHUNKS_PER_W // NHALF
EDGES_PER_W = CHUNKS_PER_W * CHUNK
E_PAD = NW * EDGES_PER_W
NP = 10240
ROWS_PER_W = NP // NS

_MESH = plsc.VectorSubcoreMesh(
    core_axis_name="c", subcore_axis_name="s", num_cores=NC, num_subcores=NS
)


def _sc_edge_pass(xl, src, dst, zeros_d, with_counts):
    def _main_phase(xl_hbm, src_hbm, dst_hbm, w, srcv, dstv, bufs, semg, acc):
        for h in range(NHALF):
            pltpu.sync_copy(src_hbm.at[w, pl.ds(h * CH_H, CH_H)], srcv)
            pltpu.sync_copy(dst_hbm.at[w, pl.ds(h * CH_H, CH_H)], dstv)
            for i in range(NBUF):
                pltpu.async_copy(xl_hbm.at[srcv.at[i]], bufs[i], semg[i])
            for j in range(CH_H):
                i = j % NBUF
                pltpu.make_async_copy(
                    xl_hbm.at[srcv.at[j]], bufs[i], semg[i]).wait()
                pltpu.sync_copy(bufs[i], acc.at[dstv.at[j]], add=True)
                if j + NBUF < CH_H:
                    pltpu.async_copy(
                        xl_hbm.at[srcv.at[j + NBUF]], bufs[i], semg[i])

    def body(xl_hbm, src_hbm, dst_hbm, zd_hbm, *rest):
        (out_hbm, srcv, dstv, r0b, r1b, acc, sg0, sg1) = rest
        c = lax.axis_index("c")
        s = lax.axis_index("s")
        w = c * NS + s
        row0 = s * ROWS_PER_W
        pltpu.sync_copy(zd_hbm, acc.at[pl.ds(row0, ROWS_PER_W)])
        plsc.subcore_barrier()

        _main_phase(xl_hbm, src_hbm, dst_hbm, w, srcv, dstv,
                    (r0b, r1b), (sg0, sg1), acc)

        plsc.subcore_barrier()
        pltpu.sync_copy(acc.at[pl.ds(row0, ROWS_PER_W)],
                        out_hbm.at[c, pl.ds(row0, ROWS_PER_W)])

    def body_counts(xl_hbm, src_hbm, dst_hbm, zd_hbm, on_hbm, *rest):
        (out_hbm, cnt_hbm, srcv, dstv, r0b, r1b, acc, sg0, sg1) = rest
        c = lax.axis_index("c")
        s = lax.axis_index("s")
        w = c * NS + s
        row0 = s * ROWS_PER_W
        pltpu.sync_copy(zd_hbm, acc.at[pl.ds(row0, ROWS_PER_W)])
        plsc.subcore_barrier()

        _main_phase(xl_hbm, src_hbm, dst_hbm, w, srcv, dstv,
                    (r0b, r1b), (sg0, sg1), acc)

        plsc.subcore_barrier()
        pltpu.sync_copy(acc.at[pl.ds(row0, ROWS_PER_W)],
                        out_hbm.at[c, pl.ds(row0, ROWS_PER_W)])
        plsc.subcore_barrier()
        pltpu.sync_copy(zd_hbm, acc.at[pl.ds(row0, ROWS_PER_W)])
        pltpu.sync_copy(on_hbm, r0b)
        plsc.subcore_barrier()

        for h in range(NHALF):
            pltpu.sync_copy(dst_hbm.at[w, pl.ds(h * CH_H, CH_H)], dstv)
            pltpu.async_copy(r0b, acc.at[dstv.at[0]], sg0, add=True)
            for j in range(1, CH_H):
                pltpu.async_copy(r0b, acc.at[dstv.at[j]], sg0, add=True)
                pltpu.make_async_copy(r0b, acc.at[dstv.at[0]], sg0).wait()
            pltpu.make_async_copy(r0b, acc.at[dstv.at[0]], sg0).wait()

        plsc.subcore_barrier()
        pltpu.sync_copy(acc.at[pl.ds(row0, ROWS_PER_W)],
                        cnt_hbm.at[c, pl.ds(row0, ROWS_PER_W)])

    scratch = (
        [pltpu.VMEM((CH_H, CHUNK), jnp.int32)] * 2
        + [pltpu.VMEM((CHUNK, DIM), jnp.float32)] * NBUF
        + [pltpu.VMEM_SHARED((NP, DIM), jnp.float32)]
        + [pltpu.SemaphoreType.DMA] * NBUF
    )
    if with_counts:
        k = pl.kernel(
            body_counts,
            out_type=(jax.ShapeDtypeStruct((NC, NP, DIM), jnp.float32),
                      jax.ShapeDtypeStruct((NC, NP, DIM), jnp.float32)),
            mesh=_MESH,
            scratch_types=scratch,
        )
        ones_d = jnp.ones((CHUNK, DIM), jnp.float32)
        return k(xl, src, dst, zeros_d, ones_d)
    k = pl.kernel(
        body,
        out_type=jax.ShapeDtypeStruct((NC, NP, DIM), jnp.float32),
        mesh=_MESH,
        scratch_types=scratch,
    )
    return k(xl, src, dst, zeros_d)


_BR = 1024


def _tc_linear(x, W, b):
    def body(x_ref, w_ref, b_ref, o_ref):
        o_ref[...] = lax.dot_general(
            x_ref[...], w_ref[...], (((1,), (1,)), ((), ())),
            preferred_element_type=jnp.float32) + b_ref[...]

    return pl.pallas_call(
        body,
        grid=(NP // _BR,),
        in_specs=[
            pl.BlockSpec((_BR, DIM), lambda i: (i, 0)),
            pl.BlockSpec((DIM, DIM), lambda i: (0, 0)),
            pl.BlockSpec((1, DIM), lambda i: (0, 0)),
        ],
        out_specs=pl.BlockSpec((_BR, DIM), lambda i: (i, 0)),
        out_shape=jax.ShapeDtypeStruct((NP, DIM), jnp.float32),
    )(x, W, b)


def _tc_combine_relu_linear(p, cp, W, b):
    def body(p_ref, cp_ref, w_ref, b_ref, o_ref, inv_ref):
        cnt = cp_ref[0, :, 0] + cp_ref[1, :, 0]
        inv = 1.0 / jnp.clip(cnt, 1.0)
        inv_ref[...] = jnp.broadcast_to(inv[:, None], (inv.shape[0], DIM))
        h = jnp.maximum((p_ref[0] + p_ref[1]) * inv[:, None], 0.0)
        o_ref[...] = lax.dot_general(
            h, w_ref[...], (((1,), (1,)), ((), ())),
            preferred_element_type=jnp.float32) + b_ref[...]

    return pl.pallas_call(
        body,
        grid=(NP // _BR,),
        in_specs=[
            pl.BlockSpec((NC, _BR, DIM), lambda i: (0, i, 0)),
            pl.BlockSpec((NC, _BR, DIM), lambda i: (0, i, 0)),
            pl.BlockSpec((DIM, DIM), lambda i: (0, 0)),
            pl.BlockSpec((1, DIM), lambda i: (0, 0)),
        ],
        out_specs=[
            pl.BlockSpec((_BR, DIM), lambda i: (i, 0)),
            pl.BlockSpec((_BR, DIM), lambda i: (i, 0)),
        ],
        out_shape=[
            jax.ShapeDtypeStruct((NP, DIM), jnp.float32),
            jax.ShapeDtypeStruct((NP, DIM), jnp.float32),
        ],
    )(p, cp, W, b)


def _tc_combine_logsoftmax(p, inv_b):
    def body(p_ref, inv_ref, h_ref, ls_ref):
        h = (p_ref[0] + p_ref[1]) * inv_ref[...]
        h_ref[...] = h
        m = jnp.max(h, axis=1, keepdims=True)
        lse = jnp.log(jnp.sum(jnp.exp(h - m), axis=1, keepdims=True)) + m
        ls_ref[...] = h - lse

    return pl.pallas_call(
        body,
        grid=(NP // _BR,),
        in_specs=[
            pl.BlockSpec((NC, _BR, DIM), lambda i: (0, i, 0)),
            pl.BlockSpec((_BR, DIM), lambda i: (i, 0)),
        ],
        out_specs=[
            pl.BlockSpec((_BR, DIM), lambda i: (i, 0)),
            pl.BlockSpec((_BR, DIM), lambda i: (i, 0)),
        ],
        out_shape=[
            jax.ShapeDtypeStruct((NP, DIM), jnp.float32),
            jax.ShapeDtypeStruct((NP, DIM), jnp.float32),
        ],
    )(p, inv_b)


def kernel(x, edge_index, W1, b1, W2, b2):
    x = x.astype(jnp.float32)
    xp = jnp.zeros((NP, DIM), jnp.float32).at[:N_NODES].set(x)
    src = edge_index[0].astype(jnp.int32)
    dst = edge_index[1].astype(jnp.int32)
    pad = E_PAD - N_EDGES
    pad_iota = jnp.arange(pad, dtype=jnp.int32)
    srcp = jnp.concatenate([src, pad_iota % N_NODES]).reshape(
        NW, CHUNKS_PER_W, CHUNK)
    dstp = jnp.concatenate([dst, N_NODES + pad_iota % (NP - N_NODES)]).reshape(
        NW, CHUNKS_PER_W, CHUNK)
    zeros_d = jnp.zeros((ROWS_PER_W, DIM), jnp.float32)
    b1r = b1.reshape(1, DIM)
    b2r = b2.reshape(1, DIM)

    xl1 = _tc_linear(xp, W1, b1r)
    p1, cp = _sc_edge_pass(xl1, srcp, dstp, zeros_d, True)
    xl2, inv_b = _tc_combine_relu_linear(p1, cp, W2, b2r)
    p2 = _sc_edge_pass(xl2, srcp, dstp, zeros_d, False)
    h2, ls = _tc_combine_logsoftmax(p2, inv_b)
    return h2[:N_NODES], ls[:N_NODES]


# speedup vs baseline: 9.7354x; 1.0005x over previous
import jax
import jax.numpy as jnp
from jax import lax
from jax.experimental import pallas as pl
from jax.experimental.pallas import tpu as pltpu
from jax.experimental.pallas import tpu_sc as plsc

N_NODES = 10000
DIM = 128
N_EDGES = 320000

NC = 2
NS = 16
NW = NC * NS
CHUNK = 128
NBUF = 2
CHUNKS_PER_W = 80
NHALF = 2
CH_H = CHUNKS_PER_W // NHALF
EDGES_PER_W = CHUNKS_PER_W * CHUNK
E_PAD = NW * EDGES_PER_W
NP = 10240
ROWS_PER_W = NP // NS

_MESH = plsc.VectorSubcoreMesh(
    core_axis_name="c", subcore_axis_name="s", num_cores=NC, num_subcores=NS
)


def _sc_edge_pass(xl, src, dst, zeros_d, with_counts):
    def _main_phase(xl_hbm, src_hbm, dst_hbm, w, srcv, dstv, bufs, semg,
                    sems, acc):
        for h in range(NHALF):
            pltpu.sync_copy(src_hbm.at[w, pl.ds(h * CH_H, CH_H)], srcv)
            pltpu.sync_copy(dst_hbm.at[w, pl.ds(h * CH_H, CH_H)], dstv)
            for i in range(NBUF):
                pltpu.async_copy(xl_hbm.at[srcv.at[i]], bufs[i], semg[i])
            for j in range(CH_H):
                i = j % NBUF
                pltpu.make_async_copy(
                    xl_hbm.at[srcv.at[j]], bufs[i], semg[i]).wait()
                pltpu.async_copy(bufs[i], acc.at[dstv.at[j]], sems[i],
                                 add=True)
                if j + NBUF < CH_H:
                    pltpu.make_async_copy(
                        bufs[i], acc.at[dstv.at[j]], sems[i]).wait()
                    pltpu.async_copy(
                        xl_hbm.at[srcv.at[j + NBUF]], bufs[i], semg[i])
            for i in range(NBUF):
                pltpu.make_async_copy(
                    bufs[i], acc.at[dstv.at[CH_H - NBUF + i]], sems[i]).wait()

    def body(xl_hbm, src_hbm, dst_hbm, zd_hbm, *rest):
        (out_hbm, srcv, dstv, r0b, r1b, acc, sg0, sg1, ss0, ss1) = rest
        c = lax.axis_index("c")
        s = lax.axis_index("s")
        w = c * NS + s
        row0 = s * ROWS_PER_W
        pltpu.sync_copy(zd_hbm, acc.at[pl.ds(row0, ROWS_PER_W)])
        plsc.subcore_barrier()

        _main_phase(xl_hbm, src_hbm, dst_hbm, w, srcv, dstv,
                    (r0b, r1b), (sg0, sg1), (ss0, ss1), acc)

        plsc.subcore_barrier()
        pltpu.sync_copy(acc.at[pl.ds(row0, ROWS_PER_W)],
                        out_hbm.at[c, pl.ds(row0, ROWS_PER_W)])

    def body_counts(xl_hbm, src_hbm, dst_hbm, zd_hbm, on_hbm, *rest):
        (out_hbm, cnt_hbm, srcv, dstv, r0b, r1b, acc, sg0, sg1, ss0, ss1) = rest
        c = lax.axis_index("c")
        s = lax.axis_index("s")
        w = c * NS + s
        row0 = s * ROWS_PER_W
        pltpu.sync_copy(zd_hbm, acc.at[pl.ds(row0, ROWS_PER_W)])
        plsc.subcore_barrier()

        _main_phase(xl_hbm, src_hbm, dst_hbm, w, srcv, dstv,
                    (r0b, r1b), (sg0, sg1), (ss0, ss1), acc)

        plsc.subcore_barrier()
        pltpu.sync_copy(acc.at[pl.ds(row0, ROWS_PER_W)],
                        out_hbm.at[c, pl.ds(row0, ROWS_PER_W)])
        plsc.subcore_barrier()
        pltpu.sync_copy(zd_hbm, acc.at[pl.ds(row0, ROWS_PER_W)])
        pltpu.sync_copy(on_hbm, r0b)
        plsc.subcore_barrier()

        for h in range(NHALF):
            pltpu.sync_copy(dst_hbm.at[w, pl.ds(h * CH_H, CH_H)], dstv)
            pltpu.async_copy(r0b, acc.at[dstv.at[0]], sg0, add=True)
            for j in range(1, CH_H):
                pltpu.async_copy(r0b, acc.at[dstv.at[j]], sg0, add=True)
                pltpu.make_async_copy(r0b, acc.at[dstv.at[0]], sg0).wait()
            pltpu.make_async_copy(r0b, acc.at[dstv.at[0]], sg0).wait()

        plsc.subcore_barrier()
        pltpu.sync_copy(acc.at[pl.ds(row0, ROWS_PER_W)],
                        cnt_hbm.at[c, pl.ds(row0, ROWS_PER_W)])

    scratch = (
        [pltpu.VMEM((CH_H, CHUNK), jnp.int32)] * 2
        + [pltpu.VMEM((CHUNK, DIM), jnp.float32)] * NBUF
        + [pltpu.VMEM_SHARED((NP, DIM), jnp.float32)]
        + [pltpu.SemaphoreType.DMA] * (2 * NBUF)
    )
    if with_counts:
        k = pl.kernel(
            body_counts,
            out_type=(jax.ShapeDtypeStruct((NC, NP, DIM), jnp.float32),
                      jax.ShapeDtypeStruct((NC, NP, DIM), jnp.float32)),
            mesh=_MESH,
            scratch_types=scratch,
        )
        ones_d = jnp.ones((CHUNK, DIM), jnp.float32)
        return k(xl, src, dst, zeros_d, ones_d)
    k = pl.kernel(
        body,
        out_type=jax.ShapeDtypeStruct((NC, NP, DIM), jnp.float32),
        mesh=_MESH,
        scratch_types=scratch,
    )
    return k(xl, src, dst, zeros_d)


_BR = 1024


def _tc_linear(x, W, b):
    def body(x_ref, w_ref, b_ref, o_ref):
        o_ref[...] = lax.dot_general(
            x_ref[...], w_ref[...], (((1,), (1,)), ((), ())),
            preferred_element_type=jnp.float32) + b_ref[...]

    return pl.pallas_call(
        body,
        grid=(NP // _BR,),
        in_specs=[
            pl.BlockSpec((_BR, DIM), lambda i: (i, 0)),
            pl.BlockSpec((DIM, DIM), lambda i: (0, 0)),
            pl.BlockSpec((1, DIM), lambda i: (0, 0)),
        ],
        out_specs=pl.BlockSpec((_BR, DIM), lambda i: (i, 0)),
        out_shape=jax.ShapeDtypeStruct((NP, DIM), jnp.float32),
    )(x, W, b)


def _tc_combine_relu_linear(p, cp, W, b):
    def body(p_ref, cp_ref, w_ref, b_ref, o_ref, inv_ref):
        cnt = cp_ref[0, :, 0] + cp_ref[1, :, 0]
        inv = 1.0 / jnp.clip(cnt, 1.0)
        inv_ref[...] = jnp.broadcast_to(inv[:, None], (inv.shape[0], DIM))
        h = jnp.maximum((p_ref[0] + p_ref[1]) * inv[:, None], 0.0)
        o_ref[...] = lax.dot_general(
            h, w_ref[...], (((1,), (1,)), ((), ())),
            preferred_element_type=jnp.float32) + b_ref[...]

    return pl.pallas_call(
        body,
        grid=(NP // _BR,),
        in_specs=[
            pl.BlockSpec((NC, _BR, DIM), lambda i: (0, i, 0)),
            pl.BlockSpec((NC, _BR, DIM), lambda i: (0, i, 0)),
            pl.BlockSpec((DIM, DIM), lambda i: (0, 0)),
            pl.BlockSpec((1, DIM), lambda i: (0, 0)),
        ],
        out_specs=[
            pl.BlockSpec((_BR, DIM), lambda i: (i, 0)),
            pl.BlockSpec((_BR, DIM), lambda i: (i, 0)),
        ],
        out_shape=[
            jax.ShapeDtypeStruct((NP, DIM), jnp.float32),
            jax.ShapeDtypeStruct((NP, DIM), jnp.float32),
        ],
    )(p, cp, W, b)


def _tc_combine_logsoftmax(p, inv_b):
    def body(p_ref, inv_ref, h_ref, ls_ref):
        h = (p_ref[0] + p_ref[1]) * inv_ref[...]
        h_ref[...] = h
        m = jnp.max(h, axis=1, keepdims=True)
        lse = jnp.log(jnp.sum(jnp.exp(h - m), axis=1, keepdims=True)) + m
        ls_ref[...] = h - lse

    return pl.pallas_call(
        body,
        grid=(NP // _BR,),
        in_specs=[
            pl.BlockSpec((NC, _BR, DIM), lambda i: (0, i, 0)),
            pl.BlockSpec((_BR, DIM), lambda i: (i, 0)),
        ],
        out_specs=[
            pl.BlockSpec((_BR, DIM), lambda i: (i, 0)),
            pl.BlockSpec((_BR, DIM), lambda i: (i, 0)),
        ],
        out_shape=[
            jax.ShapeDtypeStruct((NP, DIM), jnp.float32),
            jax.ShapeDtypeStruct((NP, DIM), jnp.float32),
        ],
    )(p, inv_b)


def kernel(x, edge_index, W1, b1, W2, b2):
    x = x.astype(jnp.float32)
    xp = jnp.zeros((NP, DIM), jnp.float32).at[:N_NODES].set(x)
    src = edge_index[0].astype(jnp.int32)
    dst = edge_index[1].astype(jnp.int32)
    pad = E_PAD - N_EDGES
    pad_iota = jnp.arange(pad, dtype=jnp.int32)
    srcp = jnp.concatenate([src, pad_iota % N_NODES]).reshape(
        NW, CHUNKS_PER_W, CHUNK)
    dstp = jnp.concatenate([dst, N_NODES + pad_iota % (NP - N_NODES)]).reshape(
        NW, CHUNKS_PER_W, CHUNK)
    zeros_d = jnp.zeros((ROWS_PER_W, DIM), jnp.float32)
    b1r = b1.reshape(1, DIM)
    b2r = b2.reshape(1, DIM)

    xl1 = _tc_linear(xp, W1, b1r)
    p1, cp = _sc_edge_pass(xl1, srcp, dstp, zeros_d, True)
    xl2, inv_b = _tc_combine_relu_linear(p1, cp, W2, b2r)
    p2 = _sc_edge_pass(xl2, srcp, dstp, zeros_d, False)
    h2, ls = _tc_combine_logsoftmax(p2, inv_b)
    return h2[:N_NODES], ls[:N_NODES]


# speedup vs baseline: 9.9834x; 1.0255x over previous
import jax
import jax.numpy as jnp
from jax import lax
from jax.experimental import pallas as pl
from jax.experimental.pallas import tpu as pltpu
from jax.experimental.pallas import tpu_sc as plsc

N_NODES = 10000
DIM = 128
N_EDGES = 320000

NC = 2
NS = 16
NW = NC * NS
CHUNK = 128
NBUF = 2
CHUNKS_PER_W = 80
NHALF = 2
CH_H = CHUNKS_PER_W // NHALF
EDGES_PER_W = CHUNKS_PER_W * CHUNK
E_PAD = NW * EDGES_PER_W
NP = 10240
ROWS_PER_W = NP // NS

_MESH = plsc.VectorSubcoreMesh(
    core_axis_name="c", subcore_axis_name="s", num_cores=NC, num_subcores=NS
)


def _sc_edge_pass(xl, src, dst, zeros_d, with_counts):
    def _main_phase(xl_hbm, src_hbm, dst_hbm, w, srcv, dstv, bufs, semg,
                    sems, acc):
        for h in range(NHALF):
            pltpu.sync_copy(src_hbm.at[w, pl.ds(h * CH_H, CH_H)], srcv)
            pltpu.sync_copy(dst_hbm.at[w, pl.ds(h * CH_H, CH_H)], dstv)
            for i in range(NBUF):
                pltpu.async_copy(xl_hbm.at[srcv.at[i]], bufs[i], semg[i])
            for j in range(CH_H):
                i = j % NBUF
                pltpu.make_async_copy(
                    xl_hbm.at[srcv.at[j]], bufs[i], semg[i]).wait()
                pltpu.async_copy(bufs[i], acc.at[dstv.at[j]], sems[i],
                                 add=True)
                if j + NBUF < CH_H:
                    pltpu.make_async_copy(
                        bufs[i], acc.at[dstv.at[j]], sems[i]).wait()
                    pltpu.async_copy(
                        xl_hbm.at[srcv.at[j + NBUF]], bufs[i], semg[i])
            for i in range(NBUF):
                pltpu.make_async_copy(
                    bufs[i], acc.at[dstv.at[CH_H - NBUF + i]], sems[i]).wait()

    def body(xl_hbm, src_hbm, dst_hbm, zd_hbm, *rest):
        (out_hbm, srcv, dstv, r0b, r1b, acc, sg0, sg1, ss0, ss1) = rest
        c = lax.axis_index("c")
        s = lax.axis_index("s")
        w = c * NS + s
        row0 = s * ROWS_PER_W
        pltpu.sync_copy(zd_hbm, acc.at[pl.ds(row0, ROWS_PER_W)])
        plsc.subcore_barrier()

        _main_phase(xl_hbm, src_hbm, dst_hbm, w, srcv, dstv,
                    (r0b, r1b), (sg0, sg1), (ss0, ss1), acc)

        plsc.subcore_barrier()
        pltpu.sync_copy(acc.at[pl.ds(row0, ROWS_PER_W)],
                        out_hbm.at[c, pl.ds(row0, ROWS_PER_W)])

    def body_counts(xl_hbm, src_hbm, dst_hbm, zd_hbm, on_hbm, *rest):
        (out_hbm, cnt_hbm, srcv, dstv, r0b, r1b, acc, sg0, sg1, ss0, ss1) = rest
        c = lax.axis_index("c")
        s = lax.axis_index("s")
        w = c * NS + s
        row0 = s * ROWS_PER_W
        pltpu.sync_copy(zd_hbm, acc.at[pl.ds(row0, ROWS_PER_W)])
        plsc.subcore_barrier()

        _main_phase(xl_hbm, src_hbm, dst_hbm, w, srcv, dstv,
                    (r0b, r1b), (sg0, sg1), (ss0, ss1), acc)

        plsc.subcore_barrier()
        pltpu.sync_copy(acc.at[pl.ds(row0, ROWS_PER_W)],
                        out_hbm.at[c, pl.ds(row0, ROWS_PER_W)])
        plsc.subcore_barrier()
        pltpu.sync_copy(zd_hbm, acc.at[pl.ds(row0, ROWS_PER_W)])
        pltpu.sync_copy(on_hbm, r0b)
        plsc.subcore_barrier()

        for h in range(NHALF):
            pltpu.sync_copy(dst_hbm.at[w, pl.ds(h * CH_H, CH_H)], dstv)
            pltpu.async_copy(r0b, acc.at[dstv.at[0]], sg0, add=True)
            for j in range(1, CH_H):
                pltpu.async_copy(r0b, acc.at[dstv.at[j]], sg0, add=True)
                pltpu.make_async_copy(r0b, acc.at[dstv.at[0]], sg0).wait()
            pltpu.make_async_copy(r0b, acc.at[dstv.at[0]], sg0).wait()

        plsc.subcore_barrier()
        pltpu.sync_copy(acc.at[pl.ds(row0, ROWS_PER_W)],
                        cnt_hbm.at[c, pl.ds(row0, ROWS_PER_W)])

    scratch = (
        [pltpu.VMEM((CH_H, CHUNK), jnp.int32)] * 2
        + [pltpu.VMEM((CHUNK, DIM), jnp.float32)] * NBUF
        + [pltpu.VMEM_SHARED((NP, DIM), jnp.float32)]
        + [pltpu.SemaphoreType.DMA] * (2 * NBUF)
    )
    if with_counts:
        k = pl.kernel(
            body_counts,
            out_type=(jax.ShapeDtypeStruct((NC, NP, DIM), jnp.float32),
                      jax.ShapeDtypeStruct((NC, NP, DIM), jnp.float32)),
            mesh=_MESH,
            scratch_types=scratch,
        )
        ones_d = jnp.ones((CHUNK, DIM), jnp.float32)
        return k(xl, src, dst, zeros_d, ones_d)
    k = pl.kernel(
        body,
        out_type=jax.ShapeDtypeStruct((NC, NP, DIM), jnp.float32),
        mesh=_MESH,
        scratch_types=scratch,
    )
    return k(xl, src, dst, zeros_d)


_BR = 1000


def _tc_linear(x, W, b):
    def body(x_ref, w_ref, b_ref, o_ref):
        o_ref[...] = lax.dot_general(
            x_ref[...], w_ref[...], (((1,), (1,)), ((), ())),
            preferred_element_type=jnp.float32) + b_ref[...]

    return pl.pallas_call(
        body,
        grid=(N_NODES // _BR,),
        in_specs=[
            pl.BlockSpec((_BR, DIM), lambda i: (i, 0)),
            pl.BlockSpec((DIM, DIM), lambda i: (0, 0)),
            pl.BlockSpec((1, DIM), lambda i: (0, 0)),
        ],
        out_specs=pl.BlockSpec((_BR, DIM), lambda i: (i, 0)),
        out_shape=jax.ShapeDtypeStruct((N_NODES, DIM), jnp.float32),
    )(x, W, b)


def _mean_from_partials(p_ref, cp_ref):
    cnt = cp_ref[0, :, 0] + cp_ref[1, :, 0]
    inv = 1.0 / jnp.clip(cnt, 1.0)
    return (p_ref[0] + p_ref[1]) * inv[:, None]


def _tc_combine_relu_linear(p, cp, W, b):
    def body(p_ref, cp_ref, w_ref, b_ref, o_ref):
        h = jnp.maximum(_mean_from_partials(p_ref, cp_ref), 0.0)
        o_ref[...] = lax.dot_general(
            h, w_ref[...], (((1,), (1,)), ((), ())),
            preferred_element_type=jnp.float32) + b_ref[...]

    return pl.pallas_call(
        body,
        grid=(N_NODES // _BR,),
        in_specs=[
            pl.BlockSpec((NC, _BR, DIM), lambda i: (0, i, 0)),
            pl.BlockSpec((NC, _BR, DIM), lambda i: (0, i, 0)),
            pl.BlockSpec((DIM, DIM), lambda i: (0, 0)),
            pl.BlockSpec((1, DIM), lambda i: (0, 0)),
        ],
        out_specs=pl.BlockSpec((_BR, DIM), lambda i: (i, 0)),
        out_shape=jax.ShapeDtypeStruct((N_NODES, DIM), jnp.float32),
    )(p, cp, W, b)


def _tc_combine_logsoftmax(p, cp):
    def body(p_ref, cp_ref, h_ref, ls_ref):
        h = _mean_from_partials(p_ref, cp_ref)
        h_ref[...] = h
        m = jnp.max(h, axis=1, keepdims=True)
        lse = jnp.log(jnp.sum(jnp.exp(h - m), axis=1, keepdims=True)) + m
        ls_ref[...] = h - lse

    return pl.pallas_call(
        body,
        grid=(N_NODES // _BR,),
        in_specs=[
            pl.BlockSpec((NC, _BR, DIM), lambda i: (0, i, 0)),
            pl.BlockSpec((NC, _BR, DIM), lambda i: (0, i, 0)),
        ],
        out_specs=[
            pl.BlockSpec((_BR, DIM), lambda i: (i, 0)),
            pl.BlockSpec((_BR, DIM), lambda i: (i, 0)),
        ],
        out_shape=[
            jax.ShapeDtypeStruct((N_NODES, DIM), jnp.float32),
            jax.ShapeDtypeStruct((N_NODES, DIM), jnp.float32),
        ],
    )(p, cp)


def kernel(x, edge_index, W1, b1, W2, b2):
    x = x.astype(jnp.float32)
    src = edge_index[0].astype(jnp.int32)
    dst = edge_index[1].astype(jnp.int32)
    pad = E_PAD - N_EDGES
    pad_iota = jnp.arange(pad, dtype=jnp.int32)
    srcp = jnp.concatenate([src, pad_iota % N_NODES]).reshape(
        NW, CHUNKS_PER_W, CHUNK)
    dstp = jnp.concatenate([dst, N_NODES + pad_iota % (NP - N_NODES)]).reshape(
        NW, CHUNKS_PER_W, CHUNK)
    zeros_d = jnp.zeros((ROWS_PER_W, DIM), jnp.float32)
    b1r = b1.reshape(1, DIM)
    b2r = b2.reshape(1, DIM)

    xl1 = _tc_linear(x, W1, b1r)
    p1, cp = _sc_edge_pass(xl1, srcp, dstp, zeros_d, True)
    xl2 = _tc_combine_relu_linear(p1, cp, W2, b2r)
    p2 = _sc_edge_pass(xl2, srcp, dstp, zeros_d, False)
    h2, ls = _tc_combine_logsoftmax(p2, cp)
    return h2, ls


# speedup vs baseline: 9.9860x; 1.0003x over previous
import jax
import jax.numpy as jnp
from jax import lax
from jax.experimental import pallas as pl
from jax.experimental.pallas import tpu as pltpu
from jax.experimental.pallas import tpu_sc as plsc

N_NODES = 10000
DIM = 128
N_EDGES = 320000

NC = 2
NS = 16
NW = NC * NS
CHUNK = 128
NBUF = 2
CHUNKS_PER_W = 80
NHALF = 2
CH_H = CHUNKS_PER_W // NHALF
EDGES_PER_W = CHUNKS_PER_W * CHUNK
E_PAD = NW * EDGES_PER_W
NP = 10240
ROWS_PER_W = NP // NS

_MESH = plsc.VectorSubcoreMesh(
    core_axis_name="c", subcore_axis_name="s", num_cores=NC, num_subcores=NS
)


def _sc_edge_pass(xl, src, dst, zeros_d, with_counts):
    def _main_phase(xl_hbm, src_hbm, dst_hbm, w, srcv, dstv, bufs, semg,
                    sems, acc):
        for h in range(NHALF):
            pltpu.sync_copy(src_hbm.at[w, pl.ds(h * CH_H, CH_H)], srcv)
            pltpu.sync_copy(dst_hbm.at[w, pl.ds(h * CH_H, CH_H)], dstv)
            for i in range(NBUF):
                pltpu.async_copy(xl_hbm.at[srcv.at[i]], bufs[i], semg[i])
            for j in range(CH_H):
                i = j % NBUF
                pltpu.make_async_copy(
                    xl_hbm.at[srcv.at[j]], bufs[i], semg[i]).wait()
                pltpu.async_copy(bufs[i], acc.at[dstv.at[j]], sems[i],
                                 add=True)
                if j + NBUF < CH_H:
                    pltpu.make_async_copy(
                        bufs[i], acc.at[dstv.at[j]], sems[i]).wait()
                    pltpu.async_copy(
                        xl_hbm.at[srcv.at[j + NBUF]], bufs[i], semg[i])
            for i in range(NBUF):
                pltpu.make_async_copy(
                    bufs[i], acc.at[dstv.at[CH_H - NBUF + i]], sems[i]).wait()

    def body(xl_hbm, src_hbm, dst_hbm, zd_hbm, *rest):
        (out_hbm, srcv, dstv, r0b, r1b, acc, sg0, sg1, ss0, ss1) = rest
        c = lax.axis_index("c")
        s = lax.axis_index("s")
        w = c * NS + s
        row0 = s * ROWS_PER_W
        pltpu.sync_copy(zd_hbm, acc.at[pl.ds(row0, ROWS_PER_W)])
        plsc.subcore_barrier()

        _main_phase(xl_hbm, src_hbm, dst_hbm, w, srcv, dstv,
                    (r0b, r1b), (sg0, sg1), (ss0, ss1), acc)

        plsc.subcore_barrier()
        pltpu.sync_copy(acc.at[pl.ds(row0, ROWS_PER_W)],
                        out_hbm.at[c, pl.ds(row0, ROWS_PER_W)])

    def body_counts(xl_hbm, src_hbm, dst_hbm, zd_hbm, on_hbm, *rest):
        (out_hbm, cnt_hbm, srcv, dstv, r0b, r1b, acc, sg0, sg1, ss0, ss1) = rest
        c = lax.axis_index("c")
        s = lax.axis_index("s")
        w = c * NS + s
        row0 = s * ROWS_PER_W
        pltpu.sync_copy(zd_hbm, acc.at[pl.ds(row0, ROWS_PER_W)])
        plsc.subcore_barrier()

        _main_phase(xl_hbm, src_hbm, dst_hbm, w, srcv, dstv,
                    (r0b, r1b), (sg0, sg1), (ss0, ss1), acc)

        plsc.subcore_barrier()
        pltpu.sync_copy(acc.at[pl.ds(row0, ROWS_PER_W)],
                        out_hbm.at[c, pl.ds(row0, ROWS_PER_W)])
        plsc.subcore_barrier()
        pltpu.sync_copy(zd_hbm, acc.at[pl.ds(row0, ROWS_PER_W)])
        pltpu.sync_copy(on_hbm, r0b)
        plsc.subcore_barrier()

        DEPTH = 4
        for h in range(NHALF):
            pltpu.sync_copy(dst_hbm.at[w, pl.ds(h * CH_H, CH_H)], dstv)
            for j in range(CH_H):
                pltpu.async_copy(r0b, acc.at[dstv.at[j]], sg0, add=True)
                if j >= DEPTH:
                    pltpu.make_async_copy(
                        r0b, acc.at[dstv.at[0]], sg0).wait()
            for _ in range(DEPTH):
                pltpu.make_async_copy(r0b, acc.at[dstv.at[0]], sg0).wait()

        plsc.subcore_barrier()
        pltpu.sync_copy(acc.at[pl.ds(row0, ROWS_PER_W)],
                        cnt_hbm.at[c, pl.ds(row0, ROWS_PER_W)])

    scratch = (
        [pltpu.VMEM((CH_H, CHUNK), jnp.int32)] * 2
        + [pltpu.VMEM((CHUNK, DIM), jnp.float32)] * NBUF
        + [pltpu.VMEM_SHARED((NP, DIM), jnp.float32)]
        + [pltpu.SemaphoreType.DMA] * (2 * NBUF)
    )
    if with_counts:
        k = pl.kernel(
            body_counts,
            out_type=(jax.ShapeDtypeStruct((NC, NP, DIM), jnp.float32),
                      jax.ShapeDtypeStruct((NC, NP, DIM), jnp.float32)),
            mesh=_MESH,
            scratch_types=scratch,
        )
        ones_d = jnp.ones((CHUNK, DIM), jnp.float32)
        return k(xl, src, dst, zeros_d, ones_d)
    k = pl.kernel(
        body,
        out_type=jax.ShapeDtypeStruct((NC, NP, DIM), jnp.float32),
        mesh=_MESH,
        scratch_types=scratch,
    )
    return k(xl, src, dst, zeros_d)


_BR = 1000


def _tc_linear(x, W, b):
    def body(x_ref, w_ref, b_ref, o_ref):
        o_ref[...] = lax.dot_general(
            x_ref[...], w_ref[...], (((1,), (1,)), ((), ())),
            preferred_element_type=jnp.float32) + b_ref[...]

    return pl.pallas_call(
        body,
        grid=(N_NODES // _BR,),
        in_specs=[
            pl.BlockSpec((_BR, DIM), lambda i: (i, 0)),
            pl.BlockSpec((DIM, DIM), lambda i: (0, 0)),
            pl.BlockSpec((1, DIM), lambda i: (0, 0)),
        ],
        out_specs=pl.BlockSpec((_BR, DIM), lambda i: (i, 0)),
        out_shape=jax.ShapeDtypeStruct((N_NODES, DIM), jnp.float32),
    )(x, W, b)


def _mean_from_partials(p_ref, cp_ref):
    cnt = cp_ref[0, :, 0] + cp_ref[1, :, 0]
    inv = 1.0 / jnp.clip(cnt, 1.0)
    return (p_ref[0] + p_ref[1]) * inv[:, None]


def _tc_combine_relu_linear(p, cp, W, b):
    def body(p_ref, cp_ref, w_ref, b_ref, o_ref):
        h = jnp.maximum(_mean_from_partials(p_ref, cp_ref), 0.0)
        o_ref[...] = lax.dot_general(
            h, w_ref[...], (((1,), (1,)), ((), ())),
            preferred_element_type=jnp.float32) + b_ref[...]

    return pl.pallas_call(
        body,
        grid=(N_NODES // _BR,),
        in_specs=[
            pl.BlockSpec((NC, _BR, DIM), lambda i: (0, i, 0)),
            pl.BlockSpec((NC, _BR, DIM), lambda i: (0, i, 0)),
            pl.BlockSpec((DIM, DIM), lambda i: (0, 0)),
            pl.BlockSpec((1, DIM), lambda i: (0, 0)),
        ],
        out_specs=pl.BlockSpec((_BR, DIM), lambda i: (i, 0)),
        out_shape=jax.ShapeDtypeStruct((N_NODES, DIM), jnp.float32),
    )(p, cp, W, b)


def _tc_combine_logsoftmax(p, cp):
    def body(p_ref, cp_ref, h_ref, ls_ref):
        h = _mean_from_partials(p_ref, cp_ref)
        h_ref[...] = h
        m = jnp.max(h, axis=1, keepdims=True)
        lse = jnp.log(jnp.sum(jnp.exp(h - m), axis=1, keepdims=True)) + m
        ls_ref[...] = h - lse

    return pl.pallas_call(
        body,
        grid=(N_NODES // _BR,),
        in_specs=[
            pl.BlockSpec((NC, _BR, DIM), lambda i: (0, i, 0)),
            pl.BlockSpec((NC, _BR, DIM), lambda i: (0, i, 0)),
        ],
        out_specs=[
            pl.BlockSpec((_BR, DIM), lambda i: (i, 0)),
            pl.BlockSpec((_BR, DIM), lambda i: (i, 0)),
        ],
        out_shape=[
            jax.ShapeDtypeStruct((N_NODES, DIM), jnp.float32),
            jax.ShapeDtypeStruct((N_NODES, DIM), jnp.float32),
        ],
    )(p, cp)


def kernel(x, edge_index, W1, b1, W2, b2):
    x = x.astype(jnp.float32)
    src = edge_index[0].astype(jnp.int32)
    dst = edge_index[1].astype(jnp.int32)
    pad = E_PAD - N_EDGES
    pad_iota = jnp.arange(pad, dtype=jnp.int32)
    srcp = jnp.concatenate([src, pad_iota % N_NODES]).reshape(
        NW, CHUNKS_PER_W, CHUNK)
    dstp = jnp.concatenate([dst, N_NODES + pad_iota % (NP - N_NODES)]).reshape(
        NW, CHUNKS_PER_W, CHUNK)
    zeros_d = jnp.zeros((ROWS_PER_W, DIM), jnp.float32)
    b1r = b1.reshape(1, DIM)
    b2r = b2.reshape(1, DIM)

    xl1 = _tc_linear(x, W1, b1r)
    p1, cp = _sc_edge_pass(xl1, srcp, dstp, zeros_d, True)
    xl2 = _tc_combine_relu_linear(p1, cp, W2, b2r)
    p2 = _sc_edge_pass(xl2, srcp, dstp, zeros_d, False)
    h2, ls = _tc_combine_logsoftmax(p2, cp)
    return h2, ls
